# Initial kernel scaffold; baseline (speedup 1.0000x reference)
import jax
import jax.numpy as jnp
from jax.experimental import pallas as pl


def kernel(x0, edge_index, batch, params):
    raise NotImplementedError("write your pallas kernel here")


# speedup vs baseline: 5.7092x; 5.7092x over previous
import functools
import numpy as np
import jax
import jax.numpy as jnp
from jax import lax
from jax.experimental import pallas as pl

_F32 = jnp.float32
_NG = 512
_HEADS = 4
_CH = 96
_DH = _CH // _HEADS
_KS = 10
_BLK = 2000
_BLK5 = 1000
_GB = 64


def _dot(a, b):
    return jnp.dot(a, b, preferred_element_type=_F32)


def _dot_t(a, b):
    return lax.dot_general(a, b, (((0,), (0,)), ((), ())),
                           preferred_element_type=_F32)


def _dot_nt(a, b):
    return lax.dot_general(a, b, (((1,), (1,)), ((), ())),
                           preferred_element_type=_F32)


def _mm_body(x_ref, w_ref, o_ref):
    o_ref[...] = _dot(x_ref[...], w_ref[...])


def _matmul(x, w, blk):
    n, fi = x.shape
    fo = w.shape[1]
    return pl.pallas_call(
        _mm_body,
        grid=(n // blk,),
        in_specs=[pl.BlockSpec((blk, fi), lambda i: (i, 0)),
                  pl.BlockSpec((fi, fo), lambda i: (0, 0))],
        out_specs=pl.BlockSpec((blk, fo), lambda i: (i, 0)),
        out_shape=jax.ShapeDtypeStruct((n, fo), _F32),
    )(x, w)


def _layer_body(a_ref, b_ref, w_ref, x_ref, h_ref):
    x = jnp.maximum(a_ref[...] + b_ref[...], 0.0)
    x_ref[...] = x
    h_ref[...] = _dot(x, w_ref[...])


def _layer(a, bias, w, blk):
    n, f = a.shape
    fo = w.shape[1]
    return pl.pallas_call(
        _layer_body,
        grid=(n // blk,),
        in_specs=[pl.BlockSpec((blk, f), lambda i: (i, 0)),
                  pl.BlockSpec((1, f), lambda i: (0, 0)),
                  pl.BlockSpec((f, fo), lambda i: (0, 0))],
        out_specs=[pl.BlockSpec((blk, f), lambda i: (i, 0)),
                   pl.BlockSpec((blk, fo), lambda i: (i, 0))],
        out_shape=[jax.ShapeDtypeStruct((n, f), _F32),
                   jax.ShapeDtypeStruct((n, fo), _F32)],
    )(a, bias, w)


def _front_body(x1_ref, x2_ref, a3_ref, b3_ref, wa_ref, wb_ref, wc_ref,
                plb_ref, wk_ref, bk_ref, wv_ref, bv_ref, qm_ref,
                s_ref, v_ref):
    x3 = jnp.maximum(a3_ref[...] + b3_ref[...], 0.0)
    y = _dot(x1_ref[...], wa_ref[...]) + _dot(x2_ref[...], wb_ref[...])
    y = y + _dot(x3, wc_ref[...]) + plb_ref[...]
    y = jnp.maximum(y, 0.0)
    k = _dot(y, wk_ref[...]) + bk_ref[...]
    v_ref[...] = _dot(y, wv_ref[...]) + bv_ref[...]
    s_ref[...] = _dot(k, qm_ref[...])


def _front(x1, x2, a3, b3, wa, wb, wc, plb, wk, bk, wv, bv, qmat, blk):
    n, f = x1.shape
    full = lambda arr: pl.BlockSpec(arr.shape, lambda i: (0,) * arr.ndim)
    return pl.pallas_call(
        _front_body,
        grid=(n // blk,),
        in_specs=[pl.BlockSpec((blk, f), lambda i: (i, 0)),
                  pl.BlockSpec((blk, f), lambda i: (i, 0)),
                  pl.BlockSpec((blk, f), lambda i: (i, 0)),
                  full(b3), full(wa), full(wb), full(wc), full(plb),
                  full(wk), full(bk), full(wv), full(bv), full(qmat)],
        out_specs=[pl.BlockSpec((blk, _HEADS * _KS), lambda i: (i, 0)),
                   pl.BlockSpec((blk, _CH), lambda i: (i, 0))],
        out_shape=[jax.ShapeDtypeStruct((n, _HEADS * _KS), _F32),
                   jax.ShapeDtypeStruct((n, _CH), _F32)],
    )(x1, x2, a3, b3, wa, wb, wc, plb, wk, bk, wv, bv, qmat)


def _seg_body(s_ref, v_ref, b_ref, r2_ref, r3_ref, num_ref, den_ref):
    @pl.when(pl.program_id(0) == 0)
    def _init():
        num_ref[...] = jnp.zeros_like(num_ref)
        den_ref[...] = jnp.zeros_like(den_ref)

    es = jnp.exp(s_ref[...])
    rep = _dot(es, r2_ref[...])
    tilev = _dot(v_ref[...], r3_ref[...])
    blk = s_ref.shape[0]
    gids = lax.broadcasted_iota(jnp.int32, (blk, _NG), 1)
    g = (gids == b_ref[...]).astype(_F32)
    num_ref[...] += _dot_t(g, rep * tilev)
    den_ref[...] += _dot_t(g, rep)


def _segment(s2, v, batch2d, r2, r3, blk):
    n = s2.shape[0]
    w = _KS * _CH
    full = lambda arr: pl.BlockSpec(arr.shape, lambda i: (0,) * arr.ndim)
    return pl.pallas_call(
        _seg_body,
        grid=(n // blk,),
        in_specs=[pl.BlockSpec((blk, _HEADS * _KS), lambda i: (i, 0)),
                  pl.BlockSpec((blk, _CH), lambda i: (i, 0)),
                  pl.BlockSpec((blk, 1), lambda i: (i, 0)),
                  full(r2), full(r3)],
        out_specs=[pl.BlockSpec((_NG, w), lambda i: (0, 0)),
                   pl.BlockSpec((_NG, w), lambda i: (0, 0))],
        out_shape=[jax.ShapeDtypeStruct((_NG, w), _F32),
                   jax.ShapeDtypeStruct((_NG, w), _F32)],
    )(s2, v, batch2d, r2, r3)


def _mha_block(q, k, v, mask, inv_sqrt):
    nk = k.shape[0]
    out = None
    for h in range(_HEADS):
        qh = q[:, h * _DH:(h + 1) * _DH]
        kh = k[:, h * _DH:(h + 1) * _DH]
        sc = _dot_nt(qh, kh) * inv_sqrt + mask
        m = jnp.max(sc, axis=-1, keepdims=True)
        e = jnp.exp(sc - m)
        a = e / jnp.sum(e, axis=-1, keepdims=True)
        lanes = lax.broadcasted_iota(jnp.int32, (1, _CH), 1)
        cm = ((lanes >= h * _DH) & (lanes < (h + 1) * _DH)).astype(_F32)
        o = _dot(a, v * cm)
        out = o if out is None else out + o
    return out


def _tail_body(num_ref, den_ref, seedt_ref, seed2t_ref, ma_ref, mb_ref,
               p1wo_ref, p1bo_ref, p1wl_ref, p1bl_ref,
               swq_ref, sbq_ref, swk_ref, sbk_ref, swv_ref, sbv_ref,
               swo_ref, sbo_ref, swl_ref, sbl_ref,
               p2lw_ref, p2lb_ref,
               p2wq_ref, p2bq_ref, p2wk_ref, p2bk_ref, p2wv_ref, p2bv_ref,
               p2wo_ref, p2bo_ref, p2wl_ref, p2bl_ref,
               l1w_ref, l1b_ref, l2w_ref, l2b_ref, o_ref):
    inv_sqrt = np.float32(1.0 / np.sqrt(_DH))
    den = den_ref[...]
    att = num_ref[...] / jnp.where(den == 0.0, 1.0, den)
    o1 = _dot(att, p1wo_ref[...]) + p1bo_ref[...] + seedt_ref[...]
    h1 = o1 + jnp.maximum(_dot(o1, p1wl_ref[...]) + p1bl_ref[...], 0.0)
    qs = _dot(h1, swq_ref[...]) + sbq_ref[...]
    ks = _dot(h1, swk_ref[...]) + sbk_ref[...]
    vs = _dot(h1, swv_ref[...]) + sbv_ref[...]
    mh = _mha_block(qs, ks, vs, ma_ref[...], inv_sqrt)
    os_ = _dot(mh, swo_ref[...]) + sbo_ref[...] + h1
    h2 = os_ + jnp.maximum(_dot(os_, swl_ref[...]) + sbl_ref[...], 0.0)
    y2 = jnp.maximum(_dot(h2, p2lw_ref[...]) + p2lb_ref[...], 0.0)
    q2 = _dot(seed2t_ref[...], p2wq_ref[...]) + p2bq_ref[...]
    k2 = _dot(y2, p2wk_ref[...]) + p2bk_ref[...]
    v2 = _dot(y2, p2wv_ref[...]) + p2bv_ref[...]
    mh2 = _mha_block(q2, k2, v2, mb_ref[...], inv_sqrt)
    o2 = _dot(mh2, p2wo_ref[...]) + p2bo_ref[...] + seed2t_ref[...]
    h3 = o2 + jnp.maximum(_dot(o2, p2wl_ref[...]) + p2bl_ref[...], 0.0)
    z = jnp.maximum(_dot(h3, l1w_ref[...]) + l1b_ref[...], 0.0)
    o_ref[...] = _dot(z, l2w_ref[...]) + l2b_ref[...]


def _tail(num_flat, den_flat, seedt, seed2t, ma, mb, weights, ncls):
    rows = _GB * _KS
    full = lambda arr: pl.BlockSpec(arr.shape, lambda i: (0,) * arr.ndim)
    in_specs = [pl.BlockSpec((rows, _CH), lambda i: (i, 0)),
                pl.BlockSpec((rows, _CH), lambda i: (i, 0)),
                full(seedt), full(seed2t), full(ma), full(mb)]
    in_specs += [full(w) for w in weights]
    return pl.pallas_call(
        _tail_body,
        grid=(_NG // _GB,),
        in_specs=in_specs,
        out_specs=pl.BlockSpec((_GB, ncls), lambda i: (i, 0)),
        out_shape=jax.ShapeDtypeStruct((_NG, ncls), _F32),
    )(num_flat, den_flat, seedt, seed2t, ma, mb, *weights)


def kernel(x0, edge_index, batch, params):
    n = x0.shape[0]
    p = params
    ar = jnp.arange(n, dtype=edge_index.dtype)
    src = jnp.concatenate([edge_index[0], ar])
    dst = jnp.concatenate([edge_index[1], ar])
    deg = jnp.zeros(n, _F32).at[dst].add(1.0)
    dinv = jnp.where(deg > 0, 1.0 / jnp.sqrt(jnp.maximum(deg, 1e-12)), 0.0)
    norm = (dinv[src] * dinv[dst])[:, None]

    def agg(h):
        return jnp.zeros_like(h).at[dst].add(h[src] * norm)

    b1 = p['b1'].reshape(1, -1)
    b2 = p['b2'].reshape(1, -1)
    b3 = p['b3'].reshape(1, -1)

    h1 = _matmul(x0, p['W1'], _BLK)
    a1 = agg(h1)
    x1, h2 = _layer(a1, b1, p['W2'], _BLK)
    a2 = agg(h2)
    x2, h3 = _layer(a2, b2, p['W3'], _BLK)
    a3 = agg(h3)

    m1 = p['pma1_mab']
    qh_full = p['pma1_seed'] @ m1['Wq'] + m1['bq']
    qmat = jnp.zeros((_CH, _HEADS * _KS), _F32)
    for h in range(_HEADS):
        blk_q = qh_full[:, h * _DH:(h + 1) * _DH].T / np.sqrt(_DH)
        qmat = qmat.at[h * _DH:(h + 1) * _DH, h * _KS:(h + 1) * _KS].set(blk_q)

    plw = p['pma1_lin_W']
    s, v = _front(x1, x2, a3, b3,
                  plw[0:32], plw[32:64], plw[64:96],
                  p['pma1_lin_b'].reshape(1, -1),
                  m1['Wk'], m1['bk'].reshape(1, -1),
                  m1['Wv'], m1['bv'].reshape(1, -1), qmat, _BLK)

    s2 = s - jnp.max(s)

    hq = np.arange(_HEADS * _KS)
    hh, qq = hq // _KS, hq % _KS
    r2 = np.zeros((_HEADS * _KS, _KS * _CH), np.float32)
    for d in range(_DH):
        r2[hq, qq * _CH + hh * _DH + d] = 1.0
    r3 = np.zeros((_CH, _KS * _CH), np.float32)
    for c in range(_CH):
        for q in range(_KS):
            r3[c, q * _CH + c] = 1.0
    r2 = jnp.asarray(r2)
    r3 = jnp.asarray(r3)

    num, den = _segment(s2, v, batch.reshape(-1, 1), r2, r3, _BLK5)
    num_flat = num.reshape(_NG * _KS, _CH)
    den_flat = den.reshape(_NG * _KS, _CH)

    rows = _GB * _KS
    seedt = jnp.tile(p['pma1_seed'], (_GB, 1))
    seed2t = jnp.tile(p['pma2_seed'], (_GB, 1))
    ri = np.arange(rows)
    ma = jnp.asarray(np.where(ri[:, None] // _KS == ri[None, :] // _KS,
                              0.0, -1e9).astype(np.float32))
    mb = jnp.asarray(np.where(ri[None, :] // _KS == np.arange(_GB)[:, None],
                              0.0, -1e9).astype(np.float32))

    m2, ms = p['pma2_mab'], p['sab_mab']
    rb = lambda b: b.reshape(1, -1)
    weights = [m1['Wo'], rb(m1['bo']), m1['Wl'], rb(m1['bl']),
               ms['Wq'], rb(ms['bq']), ms['Wk'], rb(ms['bk']),
               ms['Wv'], rb(ms['bv']), ms['Wo'], rb(ms['bo']),
               ms['Wl'], rb(ms['bl']),
               p['pma2_lin_W'], rb(p['pma2_lin_b']),
               m2['Wq'], rb(m2['bq']), m2['Wk'], rb(m2['bk']),
               m2['Wv'], rb(m2['bv']), m2['Wo'], rb(m2['bo']),
               m2['Wl'], rb(m2['bl']),
               p['lin1_W'], rb(p['lin1_b']), p['lin2_W'], rb(p['lin2_b'])]
    ncls = p['lin2_W'].shape[1]
    return _tail(num_flat, den_flat, seedt, seed2t, ma, mb, weights, ncls)
